# baseline (speedup 1.0000x reference)
import functools

import jax
import jax.numpy as jnp
from jax import lax
from jax.experimental import pallas as pl
from jax.experimental.pallas import tpu as pltpu
from jax.experimental.pallas import tpu_sc as plsc

N, E, D = 10000, 320000, 128
NC, NS = 2, 16
NW = NC * NS
EPW = E // NW
CH = 80
NCHUNK = EPW // CH
RB = 624
TOFF = NS * RB
TAIL = N - TOFF
IDXB = 25
NBLK = NCHUNK // IDXB
CHE = 128
NCHE = 78
EMAIN = NCHE * CHE
ETAIL = EPW - EMAIN
NB = 1000
F32 = jnp.float32


def _mesh():
    return plsc.VectorSubcoreMesh(core_axis_name="c", subcore_axis_name="s")


def _deg(dstm):
    @functools.partial(
        pl.kernel,
        out_type=jax.ShapeDtypeStruct((NC, N, D), F32),
        mesh=_mesh(),
        scratch_types=[
            pltpu.VMEM((NCHUNK, CH), jnp.int32),
            pltpu.VMEM((CH, D), F32),
            pltpu.VMEM_SHARED((N, D), F32),
            pltpu.SemaphoreType.DMA,
        ],
    )
    def k(dstm_hbm, out_hbm, didx, buf, acc, sem):
        cid = lax.axis_index("c")
        sid = lax.axis_index("s")
        wid = cid * NS + sid
        ones = jnp.ones((16,), F32)
        zeros = jnp.zeros((16,), F32)

        def fill(val):
            def fill_row(r, _):
                for t in range(D // 16):
                    buf[r, pl.ds(t * 16, 16)] = val
                return 0

            lax.fori_loop(0, CH, fill_row, 0)

        fill(zeros)
        for q in range(7):
            pltpu.sync_copy(buf, acc.at[pl.ds(sid * RB + q * CH, CH)])
        pltpu.sync_copy(
            buf.at[pl.ds(0, RB - 7 * CH)],
            acc.at[pl.ds(sid * RB + 7 * CH, RB - 7 * CH)],
        )

        @pl.when(sid == 0)
        def _():
            pltpu.sync_copy(buf.at[pl.ds(0, TAIL)], acc.at[pl.ds(TOFF, TAIL)])

        pltpu.sync_copy(dstm_hbm.at[wid], didx)
        fill(ones)
        plsc.subcore_barrier()

        def body(j, _):
            pltpu.async_copy(buf, acc.at[didx.at[j]], sem, add=True)
            return 0

        lax.fori_loop(0, NCHUNK, body, 0)

        def drain(j, _):
            pltpu.make_async_copy(buf, acc.at[didx.at[0]], sem).wait()
            return 0

        lax.fori_loop(0, NCHUNK, drain, 0)
        plsc.subcore_barrier()
        pltpu.sync_copy(
            acc.at[pl.ds(sid * RB, RB)], out_hbm.at[cid, pl.ds(sid * RB, RB)]
        )

        @pl.when(sid == 0)
        def _():
            pltpu.sync_copy(
                acc.at[pl.ds(TOFF, TAIL)], out_hbm.at[cid, pl.ds(TOFF, TAIL)]
            )

    return k(dstm)


def _conv(z, srcm4, dstm4):
    @functools.partial(
        pl.kernel,
        out_type=jax.ShapeDtypeStruct((NC, N, D), F32),
        mesh=_mesh(),
        scratch_types=[
            pltpu.VMEM((IDXB, CH), jnp.int32),
            pltpu.VMEM((IDXB, CH), jnp.int32),
            pltpu.VMEM((CH, D), F32),
            pltpu.VMEM((CH, D), F32),
            pltpu.VMEM((CH, D), F32),
            pltpu.VMEM_SHARED((N, D), F32),
            pltpu.SemaphoreType.DMA,
            pltpu.SemaphoreType.DMA,
            pltpu.SemaphoreType.DMA,
            pltpu.SemaphoreType.DMA,
            pltpu.SemaphoreType.DMA,
            pltpu.SemaphoreType.DMA,
        ],
    )
    def k(z_hbm, srcm_hbm, dstm_hbm, out_hbm, sidx, didx, r0, r1, r2, acc,
          g0, g1, g2, s0, s1, s2):
        cid = lax.axis_index("c")
        sid = lax.axis_index("s")
        wid = cid * NS + sid
        rows = (r0, r1, r2)
        gsem = (g0, g1, g2)
        ssem = (s0, s1, s2)
        pltpu.sync_copy(z_hbm.at[pl.ds(sid * RB, RB)], acc.at[pl.ds(sid * RB, RB)])

        @pl.when(sid == 0)
        def _():
            pltpu.sync_copy(z_hbm.at[pl.ds(TOFF, TAIL)], acc.at[pl.ds(TOFF, TAIL)])

        plsc.subcore_barrier()

        def gather(j, p):
            pltpu.async_copy(z_hbm.at[sidx.at[j]], rows[p], gsem[p])

        def gwait(p):
            pltpu.make_async_copy(z_hbm.at[sidx.at[0]], rows[p], gsem[p]).wait()

        def scatter(j, p):
            pltpu.async_copy(rows[p], acc.at[didx.at[j]], ssem[p], add=True)

        def swait(p):
            pltpu.make_async_copy(rows[p], acc.at[didx.at[0]], ssem[p]).wait()

        def step(j, p):
            gwait(p)
            scatter(j, p)

            @pl.when(j + 3 < IDXB)
            def _():
                swait(p)
                gather(j + 3, p)

        def trip(jo, _):
            step(3 * jo, 0)
            step(3 * jo + 1, 1)
            step(3 * jo + 2, 2)
            return 0

        for blk in range(NBLK):
            pltpu.sync_copy(srcm_hbm.at[wid, blk], sidx)
            pltpu.sync_copy(dstm_hbm.at[wid, blk], didx)
            gather(0, 0)
            gather(1, 1)
            gather(2, 2)
            lax.fori_loop(0, IDXB // 3, trip, 0)
            step(IDXB - 1, 0)
            swait(0)
            swait(1)
            swait(2)

        plsc.subcore_barrier()
        pltpu.sync_copy(
            acc.at[pl.ds(sid * RB, RB)], out_hbm.at[cid, pl.ds(sid * RB, RB)]
        )

        @pl.when(sid == 0)
        def _():
            pltpu.sync_copy(
                acc.at[pl.ds(TOFF, TAIL)], out_hbm.at[cid, pl.ds(TOFF, TAIL)]
            )

    return k(z, srcm4, dstm4)


def _edge(P, Q, srcm_e, dstm_e, src_t, dst_t):
    @functools.partial(
        pl.kernel,
        out_type=jax.ShapeDtypeStruct((E, D), F32),
        mesh=_mesh(),
        scratch_types=[
            pltpu.VMEM((NCHE, CHE), jnp.int32),
            pltpu.VMEM((NCHE, CHE), jnp.int32),
            pltpu.VMEM((16,), jnp.int32),
            pltpu.VMEM((16,), jnp.int32),
            pltpu.VMEM((CHE, D), F32),
            pltpu.VMEM((CHE, D), F32),
            pltpu.VMEM((CHE, D), F32),
            pltpu.VMEM((CHE, D), F32),
            pltpu.VMEM((CHE, D), F32),
            pltpu.VMEM((CHE, D), F32),
            pltpu.SemaphoreType.DMA,
            pltpu.SemaphoreType.DMA,
            pltpu.SemaphoreType.DMA,
            pltpu.SemaphoreType.DMA,
            pltpu.SemaphoreType.DMA,
            pltpu.SemaphoreType.DMA,
        ],
    )
    def k(p_hbm, q_hbm, srcm_hbm, dstm_hbm, srct_hbm, dstt_hbm, out_hbm,
          sidx, didx, sidxt, didxt, a0, a1, b0, b1, o0, o1,
          ga0, ga1, gb0, gb1, w0, w1):
        cid = lax.axis_index("c")
        sid = lax.axis_index("s")
        wid = cid * NS + sid
        base0 = wid * EMAIN
        tbase = NW * EMAIN + wid * ETAIL
        bufa = (a0, a1)
        bufb = (b0, b1)
        bufo = (o0, o1)
        gsa = (ga0, ga1)
        gsb = (gb0, gb1)
        wsem = (w0, w1)
        pltpu.sync_copy(srcm_hbm.at[wid], sidx)
        pltpu.sync_copy(dstm_hbm.at[wid], didx)
        pltpu.sync_copy(srct_hbm.at[wid, 0], sidxt)
        pltpu.sync_copy(dstt_hbm.at[wid, 0], didxt)

        def gather(j, p):
            pltpu.async_copy(p_hbm.at[sidx.at[j]], bufa[p], gsa[p])
            pltpu.async_copy(q_hbm.at[didx.at[j]], bufb[p], gsb[p])

        def gwait(p):
            pltpu.make_async_copy(p_hbm.at[sidx.at[0]], bufa[p], gsa[p]).wait()
            pltpu.make_async_copy(q_hbm.at[didx.at[0]], bufb[p], gsb[p]).wait()

        def wwait(p):
            pltpu.make_async_copy(
                bufo[p], out_hbm.at[pl.ds(base0, CHE)], wsem[p]
            ).wait()

        gather(0, 0)
        gather(1, 1)

        def add_rows(p, nrows):
            def row(r, _):
                for t in range(D // 16):
                    sl = pl.ds(t * 16, 16)
                    bufo[p][r, sl] = bufa[p][r, sl] + bufb[p][r, sl]
                return 0

            lax.fori_loop(0, nrows, row, 0)

        def step(j, p):
            gwait(p)

            @pl.when(j >= 2)
            def _():
                wwait(p)

            add_rows(p, CHE)

            @pl.when(j + 2 < NCHE)
            def _():
                gather(j + 2, p)

            pltpu.async_copy(
                bufo[p], out_hbm.at[pl.ds(base0 + j * CHE, CHE)], wsem[p]
            )

        def pair(jo, _):
            step(2 * jo, 0)
            step(2 * jo + 1, 1)
            return 0

        lax.fori_loop(0, NCHE // 2, pair, 0)
        wwait(0)
        wwait(1)
        sa = pltpu.async_copy(p_hbm.at[sidxt], a0.at[pl.ds(0, 16)], ga0)
        sb = pltpu.async_copy(q_hbm.at[didxt], b0.at[pl.ds(0, 16)], gb0)
        sa.wait()
        sb.wait()
        add_rows(0, 16)
        pltpu.sync_copy(o0.at[pl.ds(0, 16)], out_hbm.at[pl.ds(tbase, 16)])

    return k(P, Q, srcm_e, dstm_e, src_t, dst_t)


def _prep(degp, x, W1):
    def body(degp_ref, x_ref, w_ref, dinv_ref, z_ref):
        deg = 1.0 + degp_ref[0, :, 0:1] + degp_ref[1, :, 0:1]
        dcol = lax.rsqrt(deg)
        dinv_ref[...] = dcol
        y = jnp.dot(x_ref[...], w_ref[...], preferred_element_type=F32)
        z_ref[...] = y * dcol

    return pl.pallas_call(
        body,
        grid=(N // NB,),
        in_specs=[
            pl.BlockSpec((NC, NB, D), lambda i: (0, i, 0)),
            pl.BlockSpec((NB, D), lambda i: (i, 0)),
            pl.BlockSpec((D, D), lambda i: (0, 0)),
        ],
        out_specs=[
            pl.BlockSpec((NB, 1), lambda i: (i, 0)),
            pl.BlockSpec((NB, D), lambda i: (i, 0)),
        ],
        out_shape=[
            jax.ShapeDtypeStruct((N, 1), F32),
            jax.ShapeDtypeStruct((N, D), F32),
        ],
    )(degp, x, W1)


def _mid(sp, z1, dinv, b1, W2):
    def body(sp_ref, z1_ref, dinv_ref, b1_ref, w_ref, z2_ref):
        s = sp_ref[0] + sp_ref[1] - z1_ref[...]
        dcol = dinv_ref[...]
        h = jnp.maximum(s * dcol + b1_ref[...], 0.0)
        y = jnp.dot(h, w_ref[...], preferred_element_type=F32)
        z2_ref[...] = y * dcol

    return pl.pallas_call(
        body,
        grid=(N // NB,),
        in_specs=[
            pl.BlockSpec((NC, NB, D), lambda i: (0, i, 0)),
            pl.BlockSpec((NB, D), lambda i: (i, 0)),
            pl.BlockSpec((NB, 1), lambda i: (i, 0)),
            pl.BlockSpec((1, D), lambda i: (0, 0)),
            pl.BlockSpec((D, D), lambda i: (0, 0)),
        ],
        out_specs=pl.BlockSpec((NB, D), lambda i: (i, 0)),
        out_shape=jax.ShapeDtypeStruct((N, D), F32),
    )(sp, z1, dinv, b1, W2)


def _head(sp, z2, dinv, b2, W_lin, b_lin):
    def body(sp_ref, z2_ref, dinv_ref, b2_ref, wl_ref, bl_ref, p_ref, q_ref):
        s = sp_ref[0] + sp_ref[1] - z2_ref[...]
        h = s * dinv_ref[...] + b2_ref[...]
        wl = wl_ref[...]
        p_ref[...] = (
            jnp.dot(h, wl[:D], preferred_element_type=F32) + bl_ref[...]
        )
        q_ref[...] = jnp.dot(h, wl[D:], preferred_element_type=F32)

    return pl.pallas_call(
        body,
        grid=(N // NB,),
        in_specs=[
            pl.BlockSpec((NC, NB, D), lambda i: (0, i, 0)),
            pl.BlockSpec((NB, D), lambda i: (i, 0)),
            pl.BlockSpec((NB, 1), lambda i: (i, 0)),
            pl.BlockSpec((1, D), lambda i: (0, 0)),
            pl.BlockSpec((2 * D, D), lambda i: (0, 0)),
            pl.BlockSpec((1, D), lambda i: (0, 0)),
        ],
        out_specs=[
            pl.BlockSpec((NB, D), lambda i: (i, 0)),
            pl.BlockSpec((NB, D), lambda i: (i, 0)),
        ],
        out_shape=[
            jax.ShapeDtypeStruct((N, D), F32),
            jax.ShapeDtypeStruct((N, D), F32),
        ],
    )(sp, z2, dinv, b2, W_lin, b_lin)


def kernel(x, edge_index, W1, b1, W2, b2, W_lin, b_lin):
    src, dst = edge_index[0], edge_index[1]
    srcm = src.reshape(NW, NCHUNK, CH)
    dstm = dst.reshape(NW, NCHUNK, CH)
    srcm4 = src.reshape(NW, NBLK, IDXB, CH)
    dstm4 = dst.reshape(NW, NBLK, IDXB, CH)
    srcm_e = src[: NW * EMAIN].reshape(NW, NCHE, CHE)
    dstm_e = dst[: NW * EMAIN].reshape(NW, NCHE, CHE)
    src_t = src[NW * EMAIN :].reshape(NW, 1, ETAIL)
    dst_t = dst[NW * EMAIN :].reshape(NW, 1, ETAIL)
    degp = _deg(dstm)
    dinv, z1 = _prep(degp, x, W1)
    s1p = _conv(z1, srcm4, dstm4)
    z2 = _mid(s1p, z1, dinv, b1.reshape(1, D), W2)
    s2p = _conv(z2, srcm4, dstm4)
    P, Q = _head(s2p, z2, dinv, b2.reshape(1, D), W_lin, b_lin.reshape(1, D))
    return _edge(P, Q, srcm_e, dstm_e, src_t, dst_t)


# speedup vs baseline: 1.0045x; 1.0045x over previous
import functools

import jax
import jax.numpy as jnp
from jax import lax
from jax.experimental import pallas as pl
from jax.experimental.pallas import tpu as pltpu
from jax.experimental.pallas import tpu_sc as plsc

N, E, D = 10000, 320000, 128
NC, NS = 2, 16
NW = NC * NS
EPW = E // NW
CH = 80
NCHUNK = EPW // CH
RB = 624
TOFF = NS * RB
TAIL = N - TOFF
IDXB = 25
NBLK = NCHUNK // IDXB
NB = 1000
F32 = jnp.float32


def _mesh():
    return plsc.VectorSubcoreMesh(core_axis_name="c", subcore_axis_name="s")


def _deg(dstm):
    @functools.partial(
        pl.kernel,
        out_type=jax.ShapeDtypeStruct((NC, N, D), F32),
        mesh=_mesh(),
        scratch_types=[
            pltpu.VMEM((NCHUNK, CH), jnp.int32),
            pltpu.VMEM((CH, D), F32),
            pltpu.VMEM_SHARED((N, D), F32),
            pltpu.SemaphoreType.DMA,
        ],
    )
    def k(dstm_hbm, out_hbm, didx, buf, acc, sem):
        cid = lax.axis_index("c")
        sid = lax.axis_index("s")
        wid = cid * NS + sid
        ones = jnp.ones((16,), F32)
        zeros = jnp.zeros((16,), F32)

        def fill(val):
            def fill_row(r, _):
                for t in range(D // 16):
                    buf[r, pl.ds(t * 16, 16)] = val
                return 0

            lax.fori_loop(0, CH, fill_row, 0)

        fill(zeros)
        for q in range(7):
            pltpu.sync_copy(buf, acc.at[pl.ds(sid * RB + q * CH, CH)])
        pltpu.sync_copy(
            buf.at[pl.ds(0, RB - 7 * CH)],
            acc.at[pl.ds(sid * RB + 7 * CH, RB - 7 * CH)],
        )

        @pl.when(sid == 0)
        def _():
            pltpu.sync_copy(buf.at[pl.ds(0, TAIL)], acc.at[pl.ds(TOFF, TAIL)])

        pltpu.sync_copy(dstm_hbm.at[wid], didx)
        fill(ones)
        plsc.subcore_barrier()

        def body(j, _):
            pltpu.async_copy(buf, acc.at[didx.at[j]], sem, add=True)
            return 0

        lax.fori_loop(0, NCHUNK, body, 0)

        def drain(j, _):
            pltpu.make_async_copy(buf, acc.at[didx.at[0]], sem).wait()
            return 0

        lax.fori_loop(0, NCHUNK, drain, 0)
        plsc.subcore_barrier()
        pltpu.sync_copy(
            acc.at[pl.ds(sid * RB, RB)], out_hbm.at[cid, pl.ds(sid * RB, RB)]
        )

        @pl.when(sid == 0)
        def _():
            pltpu.sync_copy(
                acc.at[pl.ds(TOFF, TAIL)], out_hbm.at[cid, pl.ds(TOFF, TAIL)]
            )

    return k(dstm)


def _conv(z, srcm4, dstm4):
    @functools.partial(
        pl.kernel,
        out_type=jax.ShapeDtypeStruct((NC, N, D), F32),
        mesh=_mesh(),
        scratch_types=[
            pltpu.VMEM((IDXB, CH), jnp.int32),
            pltpu.VMEM((IDXB, CH), jnp.int32),
            pltpu.VMEM((CH, D), F32),
            pltpu.VMEM((CH, D), F32),
            pltpu.VMEM((CH, D), F32),
            pltpu.VMEM_SHARED((N, D), F32),
            pltpu.SemaphoreType.DMA,
            pltpu.SemaphoreType.DMA,
            pltpu.SemaphoreType.DMA,
            pltpu.SemaphoreType.DMA,
            pltpu.SemaphoreType.DMA,
            pltpu.SemaphoreType.DMA,
        ],
    )
    def k(z_hbm, srcm_hbm, dstm_hbm, out_hbm, sidx, didx, r0, r1, r2, acc,
          g0, g1, g2, s0, s1, s2):
        cid = lax.axis_index("c")
        sid = lax.axis_index("s")
        wid = cid * NS + sid
        rows = (r0, r1, r2)
        gsem = (g0, g1, g2)
        ssem = (s0, s1, s2)
        pltpu.sync_copy(z_hbm.at[pl.ds(sid * RB, RB)], acc.at[pl.ds(sid * RB, RB)])

        @pl.when(sid == 0)
        def _():
            pltpu.sync_copy(z_hbm.at[pl.ds(TOFF, TAIL)], acc.at[pl.ds(TOFF, TAIL)])

        plsc.subcore_barrier()

        def gather(j, p):
            pltpu.async_copy(z_hbm.at[sidx.at[j]], rows[p], gsem[p])

        def gwait(p):
            pltpu.make_async_copy(z_hbm.at[sidx.at[0]], rows[p], gsem[p]).wait()

        def scatter(j, p):
            pltpu.async_copy(rows[p], acc.at[didx.at[j]], ssem[p], add=True)

        def swait(p):
            pltpu.make_async_copy(rows[p], acc.at[didx.at[0]], ssem[p]).wait()

        def step(j, p):
            gwait(p)
            scatter(j, p)

            @pl.when(j + 3 < IDXB)
            def _():
                swait(p)
                gather(j + 3, p)

        def trip(jo, _):
            step(3 * jo, 0)
            step(3 * jo + 1, 1)
            step(3 * jo + 2, 2)
            return 0

        for blk in range(NBLK):
            pltpu.sync_copy(srcm_hbm.at[wid, blk], sidx)
            pltpu.sync_copy(dstm_hbm.at[wid, blk], didx)
            gather(0, 0)
            gather(1, 1)
            gather(2, 2)
            lax.fori_loop(0, IDXB // 3, trip, 0)
            step(IDXB - 1, 0)
            swait(0)
            swait(1)
            swait(2)

        plsc.subcore_barrier()
        pltpu.sync_copy(
            acc.at[pl.ds(sid * RB, RB)], out_hbm.at[cid, pl.ds(sid * RB, RB)]
        )

        @pl.when(sid == 0)
        def _():
            pltpu.sync_copy(
                acc.at[pl.ds(TOFF, TAIL)], out_hbm.at[cid, pl.ds(TOFF, TAIL)]
            )

    return k(z, srcm4, dstm4)


def _edge(P, Q, srcm4, dstm4):
    @functools.partial(
        pl.kernel,
        out_type=jax.ShapeDtypeStruct((E, D), F32),
        mesh=_mesh(),
        scratch_types=[
            pltpu.VMEM((IDXB, CH), jnp.int32),
            pltpu.VMEM((IDXB, CH), jnp.int32),
            pltpu.VMEM((CH, D), F32),
            pltpu.VMEM((CH, D), F32),
            pltpu.VMEM((CH, D), F32),
            pltpu.VMEM((CH, D), F32),
            pltpu.VMEM_SHARED((N, D), F32),
            pltpu.SemaphoreType.DMA,
            pltpu.SemaphoreType.DMA,
            pltpu.SemaphoreType.DMA,
            pltpu.SemaphoreType.DMA,
            pltpu.SemaphoreType.DMA,
            pltpu.SemaphoreType.DMA,
        ],
    )
    def k(p_hbm, q_hbm, srcm_hbm, dstm_hbm, out_hbm, sidx, didx,
          a0, a1, b0, b1, p_sp, ga0, ga1, gb0, gb1, w0, w1):
        cid = lax.axis_index("c")
        sid = lax.axis_index("s")
        wid = cid * NS + sid
        base0 = wid * EPW
        bufa = (a0, a1)
        bufb = (b0, b1)
        gsa = (ga0, ga1)
        gsb = (gb0, gb1)
        wsem = (w0, w1)
        pltpu.sync_copy(p_hbm.at[pl.ds(sid * RB, RB)], p_sp.at[pl.ds(sid * RB, RB)])

        @pl.when(sid == 0)
        def _():
            pltpu.sync_copy(p_hbm.at[pl.ds(TOFF, TAIL)], p_sp.at[pl.ds(TOFF, TAIL)])

        plsc.subcore_barrier()

        def gather(j, p):
            pltpu.async_copy(p_sp.at[sidx.at[j]], bufa[p], gsa[p])
            pltpu.async_copy(q_hbm.at[didx.at[j]], bufb[p], gsb[p])

        def gwait(p):
            pltpu.make_async_copy(p_sp.at[sidx.at[0]], bufa[p], gsa[p]).wait()
            pltpu.make_async_copy(q_hbm.at[didx.at[0]], bufb[p], gsb[p]).wait()

        def wwait(p):
            pltpu.make_async_copy(
                bufa[p], out_hbm.at[pl.ds(base0, CH)], wsem[p]
            ).wait()

        def add_rows(p, nrows):
            def row(r, _):
                for t in range(D // 16):
                    sl = pl.ds(t * 16, 16)
                    plsc.addupdate(bufa[p].at[r, sl], bufb[p][r, sl])
                return 0

            lax.fori_loop(0, nrows, row, 0)

        def step(j, jblk, p):
            gwait(p)
            add_rows(p, CH)
            pltpu.async_copy(
                bufa[p], out_hbm.at[pl.ds(base0 + j * CH, CH)], wsem[p]
            )

            @pl.when(jblk + 2 < IDXB)
            def _():
                wwait(p)
                gather(jblk + 2, p)

        for blk in range(NBLK):
            pltpu.sync_copy(srcm_hbm.at[wid, blk], sidx)
            pltpu.sync_copy(dstm_hbm.at[wid, blk], didx)
            gather(0, 0)
            gather(1, 1)

            def pair(jo, _):
                jb = 2 * jo
                step(blk * IDXB + jb, jb, 0)
                step(blk * IDXB + jb + 1, jb + 1, 1)
                return 0

            lax.fori_loop(0, IDXB // 2, pair, 0)
            step(blk * IDXB + IDXB - 1, IDXB - 1, 0)
            wwait(0)
            wwait(1)

    return k(P, Q, srcm4, dstm4)


def _prep(degp, x, W1):
    def body(degp_ref, x_ref, w_ref, dinv_ref, z_ref):
        deg = 1.0 + degp_ref[0, :, 0:1] + degp_ref[1, :, 0:1]
        dcol = lax.rsqrt(deg)
        dinv_ref[...] = dcol
        y = jnp.dot(x_ref[...], w_ref[...], preferred_element_type=F32)
        z_ref[...] = y * dcol

    return pl.pallas_call(
        body,
        grid=(N // NB,),
        in_specs=[
            pl.BlockSpec((NC, NB, D), lambda i: (0, i, 0)),
            pl.BlockSpec((NB, D), lambda i: (i, 0)),
            pl.BlockSpec((D, D), lambda i: (0, 0)),
        ],
        out_specs=[
            pl.BlockSpec((NB, 1), lambda i: (i, 0)),
            pl.BlockSpec((NB, D), lambda i: (i, 0)),
        ],
        out_shape=[
            jax.ShapeDtypeStruct((N, 1), F32),
            jax.ShapeDtypeStruct((N, D), F32),
        ],
    )(degp, x, W1)


def _mid(sp, z1, dinv, b1, W2):
    def body(sp_ref, z1_ref, dinv_ref, b1_ref, w_ref, z2_ref):
        s = sp_ref[0] + sp_ref[1] - z1_ref[...]
        dcol = dinv_ref[...]
        h = jnp.maximum(s * dcol + b1_ref[...], 0.0)
        y = jnp.dot(h, w_ref[...], preferred_element_type=F32)
        z2_ref[...] = y * dcol

    return pl.pallas_call(
        body,
        grid=(N // NB,),
        in_specs=[
            pl.BlockSpec((NC, NB, D), lambda i: (0, i, 0)),
            pl.BlockSpec((NB, D), lambda i: (i, 0)),
            pl.BlockSpec((NB, 1), lambda i: (i, 0)),
            pl.BlockSpec((1, D), lambda i: (0, 0)),
            pl.BlockSpec((D, D), lambda i: (0, 0)),
        ],
        out_specs=pl.BlockSpec((NB, D), lambda i: (i, 0)),
        out_shape=jax.ShapeDtypeStruct((N, D), F32),
    )(sp, z1, dinv, b1, W2)


def _head(sp, z2, dinv, b2, W_lin, b_lin):
    def body(sp_ref, z2_ref, dinv_ref, b2_ref, wl_ref, bl_ref, p_ref, q_ref):
        s = sp_ref[0] + sp_ref[1] - z2_ref[...]
        h = s * dinv_ref[...] + b2_ref[...]
        wl = wl_ref[...]
        p_ref[...] = (
            jnp.dot(h, wl[:D], preferred_element_type=F32) + bl_ref[...]
        )
        q_ref[...] = jnp.dot(h, wl[D:], preferred_element_type=F32)

    return pl.pallas_call(
        body,
        grid=(N // NB,),
        in_specs=[
            pl.BlockSpec((NC, NB, D), lambda i: (0, i, 0)),
            pl.BlockSpec((NB, D), lambda i: (i, 0)),
            pl.BlockSpec((NB, 1), lambda i: (i, 0)),
            pl.BlockSpec((1, D), lambda i: (0, 0)),
            pl.BlockSpec((2 * D, D), lambda i: (0, 0)),
            pl.BlockSpec((1, D), lambda i: (0, 0)),
        ],
        out_specs=[
            pl.BlockSpec((NB, D), lambda i: (i, 0)),
            pl.BlockSpec((NB, D), lambda i: (i, 0)),
        ],
        out_shape=[
            jax.ShapeDtypeStruct((N, D), F32),
            jax.ShapeDtypeStruct((N, D), F32),
        ],
    )(sp, z2, dinv, b2, W_lin, b_lin)


def kernel(x, edge_index, W1, b1, W2, b2, W_lin, b_lin):
    srcm = edge_index[0].reshape(NW, NCHUNK, CH)
    dstm = edge_index[1].reshape(NW, NCHUNK, CH)
    srcm4 = edge_index[0].reshape(NW, NBLK, IDXB, CH)
    dstm4 = edge_index[1].reshape(NW, NBLK, IDXB, CH)
    degp = _deg(dstm)
    dinv, z1 = _prep(degp, x, W1)
    s1p = _conv(z1, srcm4, dstm4)
    z2 = _mid(s1p, z1, dinv, b1.reshape(1, D), W2)
    s2p = _conv(z2, srcm4, dstm4)
    P, Q = _head(s2p, z2, dinv, b2.reshape(1, D), W_lin, b_lin.reshape(1, D))
    return _edge(P, Q, srcm4, dstm4)


# speedup vs baseline: 1.0048x; 1.0002x over previous
import functools

import jax
import jax.numpy as jnp
from jax import lax
from jax.experimental import pallas as pl
from jax.experimental.pallas import tpu as pltpu
from jax.experimental.pallas import tpu_sc as plsc

N, E, D = 10000, 320000, 128
NC, NS = 2, 16
NW = NC * NS
EPW = E // NW
CH = 80
NCHUNK = EPW // CH
RB = 624
TOFF = NS * RB
TAIL = N - TOFF
IDXB = 25
NBLK = NCHUNK // IDXB
NB = 1000
F32 = jnp.float32


def _mesh():
    return plsc.VectorSubcoreMesh(core_axis_name="c", subcore_axis_name="s")


def _deg(dstm):
    @functools.partial(
        pl.kernel,
        out_type=jax.ShapeDtypeStruct((NC, N, D), F32),
        mesh=_mesh(),
        scratch_types=[
            pltpu.VMEM((NCHUNK, CH), jnp.int32),
            pltpu.VMEM((CH, D), F32),
            pltpu.VMEM_SHARED((N, D), F32),
            pltpu.SemaphoreType.DMA,
        ],
    )
    def k(dstm_hbm, out_hbm, didx, buf, acc, sem):
        cid = lax.axis_index("c")
        sid = lax.axis_index("s")
        wid = cid * NS + sid
        ones = jnp.ones((16,), F32)
        zeros = jnp.zeros((16,), F32)

        def fill(val):
            def fill_row(r, _):
                for t in range(D // 16):
                    buf[r, pl.ds(t * 16, 16)] = val
                return 0

            lax.fori_loop(0, CH, fill_row, 0)

        fill(zeros)
        for q in range(7):
            pltpu.sync_copy(buf, acc.at[pl.ds(sid * RB + q * CH, CH)])
        pltpu.sync_copy(
            buf.at[pl.ds(0, RB - 7 * CH)],
            acc.at[pl.ds(sid * RB + 7 * CH, RB - 7 * CH)],
        )

        @pl.when(sid == 0)
        def _():
            pltpu.sync_copy(buf.at[pl.ds(0, TAIL)], acc.at[pl.ds(TOFF, TAIL)])

        pltpu.sync_copy(dstm_hbm.at[wid], didx)
        fill(ones)
        plsc.subcore_barrier()

        def body(j, _):
            pltpu.async_copy(buf, acc.at[didx.at[j]], sem, add=True)
            return 0

        lax.fori_loop(0, NCHUNK, body, 0)

        def drain(j, _):
            pltpu.make_async_copy(buf, acc.at[didx.at[0]], sem).wait()
            return 0

        lax.fori_loop(0, NCHUNK, drain, 0)
        plsc.subcore_barrier()
        pltpu.sync_copy(
            acc.at[pl.ds(sid * RB, RB)], out_hbm.at[cid, pl.ds(sid * RB, RB)]
        )

        @pl.when(sid == 0)
        def _():
            pltpu.sync_copy(
                acc.at[pl.ds(TOFF, TAIL)], out_hbm.at[cid, pl.ds(TOFF, TAIL)]
            )

    return k(dstm)


def _conv(z, srcm4, dstm4):
    @functools.partial(
        pl.kernel,
        out_type=jax.ShapeDtypeStruct((NC, N, D), F32),
        mesh=_mesh(),
        scratch_types=[
            pltpu.VMEM((IDXB, CH), jnp.int32),
            pltpu.VMEM((IDXB, CH), jnp.int32),
            pltpu.VMEM((CH, D), F32),
            pltpu.VMEM((CH, D), F32),
            pltpu.VMEM((CH, D), F32),
            pltpu.VMEM_SHARED((N, D), F32),
            pltpu.SemaphoreType.DMA,
            pltpu.SemaphoreType.DMA,
            pltpu.SemaphoreType.DMA,
            pltpu.SemaphoreType.DMA,
            pltpu.SemaphoreType.DMA,
            pltpu.SemaphoreType.DMA,
        ],
    )
    def k(z_hbm, srcm_hbm, dstm_hbm, out_hbm, sidx, didx, r0, r1, r2, acc,
          g0, g1, g2, s0, s1, s2):
        cid = lax.axis_index("c")
        sid = lax.axis_index("s")
        wid = cid * NS + sid
        rows = (r0, r1, r2)
        gsem = (g0, g1, g2)
        ssem = (s0, s1, s2)
        pltpu.sync_copy(z_hbm.at[pl.ds(sid * RB, RB)], acc.at[pl.ds(sid * RB, RB)])

        @pl.when(sid == 0)
        def _():
            pltpu.sync_copy(z_hbm.at[pl.ds(TOFF, TAIL)], acc.at[pl.ds(TOFF, TAIL)])

        plsc.subcore_barrier()

        def gather(j, p):
            pltpu.async_copy(z_hbm.at[sidx.at[j]], rows[p], gsem[p])

        def gwait(p):
            pltpu.make_async_copy(z_hbm.at[sidx.at[0]], rows[p], gsem[p]).wait()

        def scatter(j, p):
            pltpu.async_copy(rows[p], acc.at[didx.at[j]], ssem[p], add=True)

        def swait(p):
            pltpu.make_async_copy(rows[p], acc.at[didx.at[0]], ssem[p]).wait()

        def step(j, p):
            gwait(p)
            scatter(j, p)

            @pl.when(j + 3 < IDXB)
            def _():
                swait(p)
                gather(j + 3, p)

        def trip(jo, _):
            step(3 * jo, 0)
            step(3 * jo + 1, 1)
            step(3 * jo + 2, 2)
            return 0

        for blk in range(NBLK):
            pltpu.sync_copy(srcm_hbm.at[wid, blk], sidx)
            pltpu.sync_copy(dstm_hbm.at[wid, blk], didx)
            gather(0, 0)
            gather(1, 1)
            gather(2, 2)
            lax.fori_loop(0, IDXB // 3, trip, 0)
            step(IDXB - 1, 0)
            swait(0)
            swait(1)
            swait(2)

        plsc.subcore_barrier()
        pltpu.sync_copy(
            acc.at[pl.ds(sid * RB, RB)], out_hbm.at[cid, pl.ds(sid * RB, RB)]
        )

        @pl.when(sid == 0)
        def _():
            pltpu.sync_copy(
                acc.at[pl.ds(TOFF, TAIL)], out_hbm.at[cid, pl.ds(TOFF, TAIL)]
            )

    return k(z, srcm4, dstm4)


def _edge(P, Q, srcm4, dstm4):
    @functools.partial(
        pl.kernel,
        out_type=jax.ShapeDtypeStruct((E, D), F32),
        mesh=_mesh(),
        scratch_types=[
            pltpu.VMEM((IDXB, CH), jnp.int32),
            pltpu.VMEM((IDXB, CH), jnp.int32),
            pltpu.VMEM((CH, D), F32),
            pltpu.VMEM((CH, D), F32),
            pltpu.VMEM((CH, D), F32),
            pltpu.VMEM((CH, D), F32),
            pltpu.VMEM_SHARED((N, D), F32),
            pltpu.SemaphoreType.DMA,
            pltpu.SemaphoreType.DMA,
            pltpu.SemaphoreType.DMA,
            pltpu.SemaphoreType.DMA,
            pltpu.SemaphoreType.DMA,
            pltpu.SemaphoreType.DMA,
        ],
    )
    def k(p_hbm, q_hbm, srcm_hbm, dstm_hbm, out_hbm, sidx, didx,
          a0, a1, b0, b1, p_sp, ga0, ga1, gb0, gb1, w0, w1):
        cid = lax.axis_index("c")
        sid = lax.axis_index("s")
        wid = cid * NS + sid
        base0 = wid * EPW
        bufa = (a0, a1)
        bufb = (b0, b1)
        gsa = (ga0, ga1)
        gsb = (gb0, gb1)
        wsem = (w0, w1)
        pltpu.sync_copy(p_hbm.at[pl.ds(sid * RB, RB)], p_sp.at[pl.ds(sid * RB, RB)])

        @pl.when(sid == 0)
        def _():
            pltpu.sync_copy(p_hbm.at[pl.ds(TOFF, TAIL)], p_sp.at[pl.ds(TOFF, TAIL)])

        plsc.subcore_barrier()

        def gather(j, p):
            pltpu.async_copy(p_sp.at[sidx.at[j]], bufa[p], gsa[p])
            pltpu.async_copy(q_hbm.at[didx.at[j]], bufb[p], gsb[p])

        def gwait(p):
            pltpu.make_async_copy(p_sp.at[sidx.at[0]], bufa[p], gsa[p]).wait()
            pltpu.make_async_copy(q_hbm.at[didx.at[0]], bufb[p], gsb[p]).wait()

        def wwait(p):
            pltpu.make_async_copy(
                bufa[p], out_hbm.at[pl.ds(base0, CH)], wsem[p]
            ).wait()

        def add_rows(p, nrows):
            def row(r, _):
                for t in range(D // 16):
                    sl = pl.ds(t * 16, 16)
                    plsc.addupdate(bufa[p].at[r, sl], bufb[p][r, sl])
                return 0

            lax.fori_loop(0, nrows, row, 0)

        def step(j, jblk, p):
            gwait(p)
            add_rows(p, CH)
            pltpu.async_copy(
                bufa[p], out_hbm.at[pl.ds(base0 + j * CH, CH)], wsem[p]
            )

            @pl.when(jblk + 2 < IDXB)
            def _():
                wwait(p)
                gather(jblk + 2, p)

        for blk in range(NBLK):
            pltpu.sync_copy(srcm_hbm.at[wid, blk], sidx)
            pltpu.sync_copy(dstm_hbm.at[wid, blk], didx)
            gather(0, 0)
            gather(1, 1)

            def pair(jo, _):
                jb = 2 * jo
                step(blk * IDXB + jb, jb, 0)
                step(blk * IDXB + jb + 1, jb + 1, 1)
                return 0

            lax.fori_loop(0, IDXB // 2, pair, 0)
            step(blk * IDXB + IDXB - 1, IDXB - 1, 0)
            wwait(0)
            wwait(1)

    return k(P, Q, srcm4, dstm4)


def _prep(degp, x, W1):
    def body(degp_ref, x_ref, w_ref, dinv_ref, z_ref):
        deg = 1.0 + degp_ref[0, :, 0:1] + degp_ref[1, :, 0:1]
        dcol = lax.rsqrt(deg)
        dinv_ref[...] = dcol
        y = jnp.dot(x_ref[...], w_ref[...], preferred_element_type=F32)
        z_ref[...] = y * dcol

    return pl.pallas_call(
        body,
        grid=(N // NB,),
        in_specs=[
            pl.BlockSpec((NC, NB, D), lambda i: (0, i, 0)),
            pl.BlockSpec((NB, D), lambda i: (i, 0)),
            pl.BlockSpec((D, D), lambda i: (0, 0)),
        ],
        out_specs=[
            pl.BlockSpec((NB, 1), lambda i: (i, 0)),
            pl.BlockSpec((NB, D), lambda i: (i, 0)),
        ],
        out_shape=[
            jax.ShapeDtypeStruct((N, 1), F32),
            jax.ShapeDtypeStruct((N, D), F32),
        ],
    )(degp, x, W1)


def _mid(sp, z1, dinv, b1, W2):
    def body(sp_ref, z1_ref, dinv_ref, b1_ref, w_ref, z2_ref):
        s = sp_ref[0] + sp_ref[1] - z1_ref[...]
        dcol = dinv_ref[...]
        h = jnp.maximum(s * dcol + b1_ref[...], 0.0)
        y = jnp.dot(h, w_ref[...], preferred_element_type=F32)
        z2_ref[...] = y * dcol

    return pl.pallas_call(
        body,
        grid=(N // NB,),
        in_specs=[
            pl.BlockSpec((NC, NB, D), lambda i: (0, i, 0)),
            pl.BlockSpec((NB, D), lambda i: (i, 0)),
            pl.BlockSpec((NB, 1), lambda i: (i, 0)),
            pl.BlockSpec((1, D), lambda i: (0, 0)),
            pl.BlockSpec((D, D), lambda i: (0, 0)),
        ],
        out_specs=pl.BlockSpec((NB, D), lambda i: (i, 0)),
        out_shape=jax.ShapeDtypeStruct((N, D), F32),
    )(sp, z1, dinv, b1, W2)


def _head(sp, z2, dinv, b2, W_lin, b_lin):
    def body(sp_ref, z2_ref, dinv_ref, b2_ref, wl_ref, bl_ref, p_ref, q_ref):
        s = sp_ref[0] + sp_ref[1] - z2_ref[...]
        h = s * dinv_ref[...] + b2_ref[...]
        wl = wl_ref[...]
        p_ref[...] = (
            jnp.dot(h, wl[:D], preferred_element_type=F32) + bl_ref[...]
        )
        q_ref[...] = jnp.dot(h, wl[D:], preferred_element_type=F32)

    return pl.pallas_call(
        body,
        grid=(N // NB,),
        in_specs=[
            pl.BlockSpec((NC, NB, D), lambda i: (0, i, 0)),
            pl.BlockSpec((NB, D), lambda i: (i, 0)),
            pl.BlockSpec((NB, 1), lambda i: (i, 0)),
            pl.BlockSpec((1, D), lambda i: (0, 0)),
            pl.BlockSpec((2 * D, D), lambda i: (0, 0)),
            pl.BlockSpec((1, D), lambda i: (0, 0)),
        ],
        out_specs=[
            pl.BlockSpec((NB, D), lambda i: (i, 0)),
            pl.BlockSpec((NB, D), lambda i: (i, 0)),
        ],
        out_shape=[
            jax.ShapeDtypeStruct((N, D), F32),
            jax.ShapeDtypeStruct((N, D), F32),
        ],
    )(sp, z2, dinv, b2, W_lin, b_lin)


def kernel(x, edge_index, W1, b1, W2, b2, W_lin, b_lin):
    dstm = edge_index[1].reshape(NW, NCHUNK, CH)
    srcm4 = edge_index[0].reshape(NW, NBLK, IDXB, CH)
    dstm4 = edge_index[1].reshape(NW, NBLK, IDXB, CH)
    degp = _deg(dstm)
    dinv, z1 = _prep(degp, x, W1)
    s1p = _conv(z1, srcm4, dstm4)
    z2 = _mid(s1p, z1, dinv, b1.reshape(1, D), W2)
    s2p = _conv(z2, srcm4, dstm4)
    P, Q = _head(s2p, z2, dinv, b2.reshape(1, D), W_lin, b_lin.reshape(1, D))
    return _edge(P, Q, srcm4, dstm4)
